# baseline (speedup 1.0000x reference)
import functools

import jax
import jax.numpy as jnp
from jax import lax
from jax.experimental import pallas as pl
from jax.experimental.pallas import tpu as pltpu
from jax.experimental.pallas import tpu_sc as plsc

N_FIELDS = 26
VOCAB = 100000
EMB_DIM = 32
BATCH = 4096

NC = 2
NS = 16
NW = NC * NS
ROWS = N_FIELDS * EMB_DIM
BW = BATCH // NW
LANES = 16
CHUNKS_PER_ROW = VOCAB // LANES

N_GROUPS = 2
GF = N_FIELDS // N_GROUPS
GROWS = GF * EMB_DIM


def _sc_gather_body(idx_hbm, tab_hbm, out_hbm, idx_v, cidx_v, staged_v,
                    ebuf_v, sem):
    wid = lax.axis_index("s") * NC + lax.axis_index("c")
    col0 = wid * BW
    pltpu.sync_copy(idx_hbm.at[:, pl.ds(col0, BW)], idx_v)
    lane = lax.iota(jnp.int32, LANES)

    def per_field(f, _):
        vs = [idx_v[f, pl.ds(g * LANES, LANES)] for g in range(BW // LANES)]
        vc16 = [v >> 4 for v in vs]
        vr = [v & 15 for v in vs]
        base0 = (f * EMB_DIM) * CHUNKS_PER_ROW
        for d in range(EMB_DIM):
            base = base0 + d * CHUNKS_PER_ROW
            for g in range(BW // LANES):
                cidx_v[d, pl.ds(g * LANES, LANES)] = vc16[g] + base
        for d in range(EMB_DIM):
            pltpu.async_copy(
                tab_hbm.at[cidx_v.at[d]],
                staged_v.at[pl.ds(d * BW, BW)],
                sem,
            )
        pltpu.make_async_copy(
            tab_hbm.at[pl.ds(0, EMB_DIM * BW)], staged_v, sem
        ).wait()
        for d in range(EMB_DIM):
            for g in range(BW // LANES):
                chunkv = d * BW + g * LANES + lane
                ebuf_v[d, pl.ds(g * LANES, LANES)] = plsc.load_gather(
                    staged_v, [chunkv, vr[g]])
        pltpu.sync_copy(
            ebuf_v, out_hbm.at[pl.ds(f * EMB_DIM, EMB_DIM), pl.ds(col0, BW)])
        return 0

    lax.fori_loop(0, GF, per_field, 0)


@functools.partial(
    pl.kernel,
    mesh=plsc.VectorSubcoreMesh(core_axis_name="c", subcore_axis_name="s"),
    compiler_params=pltpu.CompilerParams(
        use_tc_tiling_on_sc=False, needs_layout_passes=False),
    out_type=jax.ShapeDtypeStruct((GROWS, BATCH), jnp.float32),
    scratch_types=[
        pltpu.VMEM((GF, BW), jnp.int32),
        pltpu.VMEM((EMB_DIM, BW), jnp.int32),
        pltpu.VMEM((EMB_DIM * BW, LANES), jnp.float32),
        pltpu.VMEM((EMB_DIM, BW), jnp.float32),
        pltpu.SemaphoreType.DMA,
    ],
)
def _sc_gather(idx_hbm, tab_hbm, out_hbm, idx_v, cidx_v, staged_v, ebuf_v,
               sem):
    _sc_gather_body(idx_hbm, tab_hbm, out_hbm, idx_v, cidx_v, staged_v,
                    ebuf_v, sem)


def _mm_body(g_ref, ws_ref, bs_ref, d_ref, wd_ref, bd_ref, so_ref, do_ref):
    so_ref[...] = lax.dot_general(
        g_ref[...], ws_ref[...], (((0,), (1,)), ((), ())),
        preferred_element_type=jnp.float32) + bs_ref[...]
    do_ref[...] = lax.dot_general(
        d_ref[...], wd_ref[...], (((1,), (1,)), ((), ())),
        preferred_element_type=jnp.float32) + bd_ref[...]


def _tc_matmuls(g, w_s, b_s, dense, w_d, b_d):
    blk = 512
    grid = (BATCH // blk,)
    d_in = dense.shape[1]
    return pl.pallas_call(
        _mm_body,
        grid=grid,
        in_specs=[
            pl.BlockSpec((ROWS, blk), lambda i: (0, i)),
            pl.BlockSpec((w_s.shape[0], ROWS), lambda i: (0, 0)),
            pl.BlockSpec((1, w_s.shape[0]), lambda i: (0, 0)),
            pl.BlockSpec((blk, d_in), lambda i: (i, 0)),
            pl.BlockSpec((w_d.shape[0], d_in), lambda i: (0, 0)),
            pl.BlockSpec((1, w_d.shape[0]), lambda i: (0, 0)),
        ],
        out_specs=[
            pl.BlockSpec((blk, w_s.shape[0]), lambda i: (i, 0)),
            pl.BlockSpec((blk, w_d.shape[0]), lambda i: (i, 0)),
        ],
        out_shape=[
            jax.ShapeDtypeStruct((BATCH, w_s.shape[0]), jnp.float32),
            jax.ShapeDtypeStruct((BATCH, w_d.shape[0]), jnp.float32),
        ],
    )(g, w_s, b_s, dense, w_d, b_d)


def kernel(sparse_inputs, dense_inputs, tables, W_sparse, b_sparse, W_dense, b_dense):
    idx_t = jnp.transpose(sparse_inputs.astype(jnp.int32), (1, 0))
    tab_native = jnp.transpose(tables, (0, 2, 1)).reshape(ROWS, VOCAB)
    gs = []
    for gi in range(N_GROUPS):
        tab_g = tab_native[gi * GROWS:(gi + 1) * GROWS, :].reshape(
            GROWS * CHUNKS_PER_ROW, LANES)
        idx_g = idx_t[gi * GF:(gi + 1) * GF, :]
        gs.append(_sc_gather(idx_g, tab_g))
    g = jnp.concatenate(gs, axis=0)
    sparse_out, dense_out = _tc_matmuls(
        g, W_sparse, b_sparse.reshape(1, -1),
        dense_inputs, W_dense, b_dense.reshape(1, -1))
    return (dense_out, sparse_out)


# speedup vs baseline: 1.3300x; 1.3300x over previous
import functools

import jax
import jax.numpy as jnp
from jax import lax
from jax.experimental import pallas as pl
from jax.experimental.pallas import tpu as pltpu
from jax.experimental.pallas import tpu_sc as plsc

N_FIELDS = 26
VOCAB = 100000
EMB_DIM = 32
BATCH = 4096

NC = 2
NS = 16
NW = NC * NS
ROWS = N_FIELDS * EMB_DIM
BW = BATCH // NW
LANES = 16
CHUNKS_PER_ROW = VOCAB // LANES

N_GROUPS = 1
GF = N_FIELDS // N_GROUPS
GROWS = GF * EMB_DIM


HD = EMB_DIM // 2
N_HALVES = GF * 2


def _sc_gather_body(idx_hbm, tab_hbm, out_hbm, idx_v, cidx0, cidx1, st0, st1,
                    ebuf_v, sem0, sem1):
    wid = lax.axis_index("s") * NC + lax.axis_index("c")
    col0 = wid * BW
    pltpu.sync_copy(idx_hbm.at[:, pl.ds(col0, BW)], idx_v)
    lane = lax.iota(jnp.int32, LANES)

    def issue(h, cidx_v, staged_v, sem):
        f = h // 2
        d0 = (h % 2) * HD
        vc16 = [idx_v[f, pl.ds(g * LANES, LANES)] >> 4
                for g in range(BW // LANES)]
        for d in range(HD):
            base = (f * EMB_DIM + d0 + d) * CHUNKS_PER_ROW
            for g in range(BW // LANES):
                cidx_v[d, pl.ds(g * LANES, LANES)] = vc16[g] + base
        for d in range(HD):
            pltpu.async_copy(
                tab_hbm.at[cidx_v.at[d]],
                staged_v.at[pl.ds(d * BW, BW)],
                sem,
            )

    def drain(staged_v, sem):
        pltpu.make_async_copy(
            tab_hbm.at[pl.ds(0, HD * BW)], staged_v, sem
        ).wait()

    def extract(h, staged_v):
        f = h // 2
        d0 = (h % 2) * HD
        vr = [idx_v[f, pl.ds(g * LANES, LANES)] & 15
              for g in range(BW // LANES)]
        for d in range(HD):
            for g in range(BW // LANES):
                chunkv = d * BW + g * LANES + lane
                ebuf_v[d, pl.ds(g * LANES, LANES)] = plsc.load_gather(
                    staged_v, [chunkv, vr[g]])
        pltpu.sync_copy(
            ebuf_v,
            out_hbm.at[pl.ds(f * EMB_DIM + d0, HD), pl.ds(col0, BW)])

    issue(0, cidx0, st0, sem0)

    def body(i, _):
        h0 = 2 * i
        issue(h0 + 1, cidx1, st1, sem1)
        drain(st0, sem0)
        extract(h0, st0)

        @pl.when(h0 + 2 < N_HALVES)
        def _():
            issue(h0 + 2, cidx0, st0, sem0)

        drain(st1, sem1)
        extract(h0 + 1, st1)
        return 0

    lax.fori_loop(0, N_HALVES // 2, body, 0)


@functools.partial(
    pl.kernel,
    mesh=plsc.VectorSubcoreMesh(core_axis_name="c", subcore_axis_name="s"),
    compiler_params=pltpu.CompilerParams(
        use_tc_tiling_on_sc=False, needs_layout_passes=False),
    out_type=jax.ShapeDtypeStruct((GROWS, BATCH), jnp.float32),
    scratch_types=[
        pltpu.VMEM((GF, BW), jnp.int32),
        pltpu.VMEM((HD, BW), jnp.int32),
        pltpu.VMEM((HD, BW), jnp.int32),
        pltpu.VMEM((HD * BW, LANES), jnp.float32),
        pltpu.VMEM((HD * BW, LANES), jnp.float32),
        pltpu.VMEM((HD, BW), jnp.float32),
        pltpu.SemaphoreType.DMA,
        pltpu.SemaphoreType.DMA,
    ],
)
def _sc_gather(idx_hbm, tab_hbm, out_hbm, idx_v, cidx0, cidx1, st0, st1,
               ebuf_v, sem0, sem1):
    _sc_gather_body(idx_hbm, tab_hbm, out_hbm, idx_v, cidx0, cidx1, st0, st1,
                    ebuf_v, sem0, sem1)


def _mm_body(g_ref, ws_ref, bs_ref, d_ref, wd_ref, bd_ref, so_ref, do_ref):
    so_ref[...] = lax.dot_general(
        g_ref[...], ws_ref[...], (((0,), (1,)), ((), ())),
        preferred_element_type=jnp.float32) + bs_ref[...]
    do_ref[...] = lax.dot_general(
        d_ref[...], wd_ref[...], (((1,), (1,)), ((), ())),
        preferred_element_type=jnp.float32) + bd_ref[...]


def _tc_matmuls(g, w_s, b_s, dense, w_d, b_d):
    blk = 512
    grid = (BATCH // blk,)
    d_in = dense.shape[1]
    return pl.pallas_call(
        _mm_body,
        grid=grid,
        in_specs=[
            pl.BlockSpec((ROWS, blk), lambda i: (0, i)),
            pl.BlockSpec((w_s.shape[0], ROWS), lambda i: (0, 0)),
            pl.BlockSpec((1, w_s.shape[0]), lambda i: (0, 0)),
            pl.BlockSpec((blk, d_in), lambda i: (i, 0)),
            pl.BlockSpec((w_d.shape[0], d_in), lambda i: (0, 0)),
            pl.BlockSpec((1, w_d.shape[0]), lambda i: (0, 0)),
        ],
        out_specs=[
            pl.BlockSpec((blk, w_s.shape[0]), lambda i: (i, 0)),
            pl.BlockSpec((blk, w_d.shape[0]), lambda i: (i, 0)),
        ],
        out_shape=[
            jax.ShapeDtypeStruct((BATCH, w_s.shape[0]), jnp.float32),
            jax.ShapeDtypeStruct((BATCH, w_d.shape[0]), jnp.float32),
        ],
    )(g, w_s, b_s, dense, w_d, b_d)


def kernel(sparse_inputs, dense_inputs, tables, W_sparse, b_sparse, W_dense, b_dense):
    idx_t = jnp.transpose(sparse_inputs.astype(jnp.int32), (1, 0))
    tab_native = jnp.transpose(tables, (0, 2, 1)).reshape(ROWS, VOCAB)
    gs = []
    for gi in range(N_GROUPS):
        tab_g = tab_native[gi * GROWS:(gi + 1) * GROWS, :].reshape(
            GROWS * CHUNKS_PER_ROW, LANES)
        idx_g = idx_t[gi * GF:(gi + 1) * GF, :]
        gs.append(_sc_gather(idx_g, tab_g))
    g = jnp.concatenate(gs, axis=0)
    sparse_out, dense_out = _tc_matmuls(
        g, W_sparse, b_sparse.reshape(1, -1),
        dense_inputs, W_dense, b_dense.reshape(1, -1))
    return (dense_out, sparse_out)
